# Initial kernel scaffold; baseline (speedup 1.0000x reference)
import jax
import jax.numpy as jnp
from jax.experimental import pallas as pl


def kernel(idx, table_pad, targets):
    raise NotImplementedError("write your pallas kernel here")


# speedup vs baseline: 1.3543x; 1.3543x over previous
import jax
import jax.numpy as jnp
from jax.experimental import pallas as pl
from jax.experimental.pallas import tpu as pltpu

_V = 256
_TILE_N = 8192


def _fwd_kernel(idx_ref, tgt_ref, table_ref, lse_ref, logits_ref, rowloss_ref):
    idx = idx_ref[...]
    tgt = tgt_ref[...]
    col = jax.lax.broadcasted_iota(jnp.int32, (1, _V), 1)

    onehot = (col == idx).astype(jnp.float32)
    logits = jnp.dot(onehot, table_ref[...],
                     preferred_element_type=jnp.float32)
    logits_ref[...] = logits

    tgt_logit = jnp.sum(jnp.where(col == tgt, logits, 0.0),
                        axis=-1, keepdims=True)
    lse_i = jnp.sum(jnp.where(col == idx, lse_ref[...], 0.0),
                    axis=-1, keepdims=True)
    rowloss_ref[...] = lse_i - tgt_logit


def kernel(idx, table_pad, targets):
    B, T = idx.shape
    N = B * T
    tn = _TILE_N
    grid = (N // tn,)

    idx_col = idx.reshape(N, 1).astype(jnp.int32)
    tgt_col = targets.reshape(N, 1).astype(jnp.int32)
    table = table_pad[:_V, :_V]
    lse_row = table_pad[:_V, _V].reshape(1, _V)

    idx_spec = pl.BlockSpec((tn, 1), lambda i: (i, 0))
    table_spec = pl.BlockSpec((_V, _V), lambda i: (0, 0))
    lse_spec = pl.BlockSpec((1, _V), lambda i: (0, 0))
    logits_spec = pl.BlockSpec((tn, _V), lambda i: (i, 0))
    rowloss_spec = pl.BlockSpec((tn, 1), lambda i: (i, 0))

    logits, rowloss = pl.pallas_call(
        _fwd_kernel,
        grid=grid,
        out_shape=(
            jax.ShapeDtypeStruct((N, _V), jnp.float32),
            jax.ShapeDtypeStruct((N, 1), jnp.float32),
        ),
        in_specs=[idx_spec, idx_spec, table_spec, lse_spec],
        out_specs=(logits_spec, rowloss_spec),
        compiler_params=pltpu.CompilerParams(
            dimension_semantics=("parallel",)),
    )(idx_col, tgt_col, table, lse_row)

    loss = jnp.sum(rowloss) * jnp.float32(1.0 / N)
    return logits, loss


# speedup vs baseline: 11.9635x; 8.8338x over previous
import jax
import jax.numpy as jnp
from jax.experimental import pallas as pl
from jax.experimental.pallas import tpu as pltpu

_V = 256
_ROWS = 8


def _fwd_kernel(idx_ref, tgt_ref, table_ref, m_ref, logits_ref, partial_ref):
    r, t = idx_ref.shape
    row = jax.lax.broadcasted_iota(jnp.int32, (_V, 1), 0)

    acc_c = jnp.zeros((_V, _V), jnp.float32)
    for j in range(r):
        irow = idx_ref[j:j + 1, :]
        trow = tgt_ref[j:j + 1, :]
        ot_idx = (row == irow).astype(jnp.float32)
        ot_tgt = (row == trow).astype(jnp.float32)

        logits_j = jax.lax.dot_general(
            ot_idx, table_ref[...],
            dimension_numbers=(((0,), (0,)), ((), ())),
            preferred_element_type=jnp.float32)
        logits_ref[pl.ds(j * t, t), :] = logits_j

        acc_c = acc_c + jax.lax.dot_general(
            ot_idx, ot_tgt,
            dimension_numbers=(((1,), (1,)), ((), ())),
            preferred_element_type=jnp.float32)

    partial = jnp.sum(acc_c * m_ref[...])
    partial_ref[...] = jnp.broadcast_to(partial, (1, 1, 128))


def kernel(idx, table_pad, targets):
    B, T = idx.shape
    N = B * T
    grid = (B // _ROWS,)

    table = table_pad[:_V, :_V]
    lse_col = table_pad[:_V, _V].reshape(_V, 1)
    m_mat = lse_col - table

    idx_spec = pl.BlockSpec((_ROWS, T), lambda i: (i, 0))
    table_spec = pl.BlockSpec((_V, _V), lambda i: (0, 0))
    logits_spec = pl.BlockSpec((_ROWS * T, _V), lambda i: (i, 0))
    partial_spec = pl.BlockSpec((1, 1, 128), lambda i: (i, 0, 0))

    logits, partials = pl.pallas_call(
        _fwd_kernel,
        grid=grid,
        out_shape=(
            jax.ShapeDtypeStruct((N, _V), jnp.float32),
            jax.ShapeDtypeStruct((grid[0], 1, 128), jnp.float32),
        ),
        in_specs=[idx_spec, idx_spec, table_spec, table_spec],
        out_specs=(logits_spec, partial_spec),
        compiler_params=pltpu.CompilerParams(
            dimension_semantics=("parallel",)),
    )(idx.astype(jnp.int32), targets.astype(jnp.int32), table, m_mat)

    loss = jnp.sum(partials[:, 0, 0]) * jnp.float32(1.0 / N)
    return logits, loss


# speedup vs baseline: 11.9984x; 1.0029x over previous
import jax
import jax.numpy as jnp
from jax.experimental import pallas as pl
from jax.experimental.pallas import tpu as pltpu

_V = 256
_ROWS = 8


def _fwd_kernel(idx_ref, tgt_ref, table_ref, logits_ref, partial_ref):
    r, t = idx_ref.shape
    row = jax.lax.broadcasted_iota(jnp.int32, (_V, 1), 0)
    table = table_ref[0:_V, 0:_V]

    acc_c = jnp.zeros((_V, _V), jnp.float32)
    for j in range(r):
        irow = idx_ref[j:j + 1, :]
        trow = tgt_ref[j:j + 1, :]
        ot_idx = (row == irow).astype(jnp.float32)
        ot_tgt = (row == trow).astype(jnp.float32)

        logits_j = jax.lax.dot_general(
            ot_idx, table,
            dimension_numbers=(((0,), (0,)), ((), ())),
            preferred_element_type=jnp.float32)
        logits_ref[pl.ds(j * t, t), :] = logits_j

        acc_c = acc_c + jax.lax.dot_general(
            ot_idx, ot_tgt,
            dimension_numbers=(((1,), (1,)), ((), ())),
            preferred_element_type=jnp.float32)

    m_mat = table_ref[0:_V, _V:_V + 1] - table
    partial = jnp.sum(acc_c * m_mat)
    partial_ref[...] = jnp.broadcast_to(partial, (1, 1, 128))


def kernel(idx, table_pad, targets):
    B, T = idx.shape
    N = B * T
    grid = (B // _ROWS,)
    v_pad = table_pad.shape[0]

    idx_spec = pl.BlockSpec((_ROWS, T), lambda i: (i, 0))
    table_spec = pl.BlockSpec((v_pad, v_pad), lambda i: (0, 0))
    logits_spec = pl.BlockSpec((_ROWS * T, _V), lambda i: (i, 0))
    partial_spec = pl.BlockSpec((1, 1, 128), lambda i: (i, 0, 0))

    logits, partials = pl.pallas_call(
        _fwd_kernel,
        grid=grid,
        out_shape=(
            jax.ShapeDtypeStruct((N, _V), jnp.float32),
            jax.ShapeDtypeStruct((grid[0], 1, 128), jnp.float32),
        ),
        in_specs=[idx_spec, idx_spec, table_spec],
        out_specs=(logits_spec, partial_spec),
        compiler_params=pltpu.CompilerParams(
            dimension_semantics=("parallel",)),
    )(idx.astype(jnp.int32), targets.astype(jnp.int32), table_pad)

    loss = jnp.sum(partials[:, 0, 0]) * jnp.float32(1.0 / N)
    return logits, loss
